# baseline (speedup 1.0000x reference)
import jax
import jax.numpy as jnp
import numpy as np
from jax import lax
from jax.experimental import pallas as pl
from jax.experimental.pallas import tpu as pltpu
from jax.experimental.pallas import tpu_sc as plsc

HIDDEN = 128
N_SYMBOLS = 100000
BATCH = 4096
SEQ = 200

NC, NS, LANES = 2, 16, 16
NW = NC * NS
FLAT = BATCH * SEQ
PER_W = FLAT // NW
NSEQ = PER_W // SEQ
VREGS = HIDDEN // LANES
NB = 4


def _pe_table() -> np.ndarray:
    power = np.arange(0, HIDDEN, 2, dtype=np.float32) / np.float32(HIDDEN)
    divisor = np.float32(10000.0) ** power
    seq_pos = np.arange(SEQ, dtype=np.float32) + np.float32(1.0)
    arg = seq_pos[:, None] / divisor[None, :]
    pe = np.empty((SEQ, HIDDEN), dtype=np.float32)
    pe[:, 0::2] = np.sin(arg)
    pe[:, 1::2] = np.cos(arg)
    return pe


_PE = _pe_table()


def _sc_body(enc_hbm, table_hbm, pe_hbm, out_hbm,
             pe_v, ovec_v, r0, r1, r2, r3, i0, i1, i2, i3, m0, m1, m2, m3,
             o0, o1, o2, o3,
             gs0, gs1, gs2, gs3, ws0, ws1, ws2, ws3, qs0, qs1, qs2, qs3):
    rows = (r0, r1, r2, r3)
    ibuf = (i0, i1, i2, i3)
    mbuf = (m0, m1, m2, m3)
    obuf = (o0, o1, o2, o3)
    gsem = (gs0, gs1, gs2, gs3)
    wsem = (ws0, ws1, ws2, ws3)
    isem = (qs0, qs1, qs2, qs3)

    wid = lax.axis_index("s") * NC + lax.axis_index("c")
    wbase = wid * PER_W
    encw = enc_hbm.at[wid]

    pltpu.sync_copy(pe_hbm, pe_v)

    for u in range(VREGS):
        sl = pl.ds(u * LANES, LANES)
        ovec_v[sl] = (lax.iota(jnp.int32, LANES) + (u * LANES)) * SEQ + wbase

    def i_start(c, k):
        pltpu.async_copy(encw.at[c], ibuf[k], isem[k])

    def i_wait(c, k):
        pltpu.make_async_copy(encw.at[c], ibuf[k], isem[k]).wait()

    def decode(k):
        for u in range(VREGS):
            sl = pl.ds(u * LANES, LANES)
            e16 = ibuf[k][sl]
            mbuf[k][sl] = jnp.where(e16 < 0, jnp.float32(1), jnp.float32(0))
            ibuf[k][sl] = e16 & jnp.int32(0x7FFFFFFF)

    def g_start(k):
        pltpu.async_copy(table_hbm.at[ibuf[k]], rows[k], gsem[k])

    def g_wait(k):
        pltpu.make_async_copy(table_hbm.at[ibuf[k]], rows[k], gsem[k]).wait()

    def w_start(k):
        pltpu.async_copy(rows[k], out_hbm.at[obuf[k]], wsem[k])

    def w_wait(k):
        pltpu.make_async_copy(rows[k], out_hbm.at[obuf[k]], wsem[k]).wait()

    def compute(c, k):
        buf = rows[k]
        for u in range(VREGS):
            sl = pl.ds(u * LANES, LANES)
            obuf[k][sl] = ovec_v[sl] + c
        p = [pe_v[c, pl.ds(v * LANES, LANES)] for v in range(VREGS)]

        @pl.loop(0, NSEQ // LANES)
        def _grp(gr):
            m16 = mbuf[k][pl.ds(gr * LANES, LANES)]
            for j16 in range(LANES):
                j = gr * LANES + j16
                m = m16[j16]
                for v in range(VREGS):
                    sl = pl.ds(v * LANES, LANES)
                    buf[j, sl] = (buf[j, sl] + p[v]) * m

    for c0 in range(NB):
        pltpu.sync_copy(encw.at[c0], ibuf[c0])
        decode(c0)
    g_start(0)
    g_start(1)

    g_wait(0)
    compute(0, 0)
    w_start(0)
    g_start(2)
    i_start(4, 0)

    g_wait(1)
    compute(1, 1)
    w_start(1)
    g_start(3)
    i_start(5, 1)

    @pl.loop(2, 194, step=4)
    def _main(go):
        for j in range(4):
            c = go + j
            k = (2 + j) % NB
            kn = j % NB
            g_wait(k)
            compute(c, k)
            w_start(k)
            w_wait(kn)
            i_wait(c + 2, kn)
            decode(kn)
            g_start(kn)
            i_start(c + 4, k)

    g_wait(2)
    compute(194, 2)
    w_start(2)
    w_wait(0)
    i_wait(196, 0)
    decode(0)
    g_start(0)
    i_start(198, 2)

    g_wait(3)
    compute(195, 3)
    w_start(3)
    w_wait(1)
    i_wait(197, 1)
    decode(1)
    g_start(1)
    i_start(199, 3)

    g_wait(0)
    compute(196, 0)
    w_start(0)
    w_wait(2)
    i_wait(198, 2)
    decode(2)
    g_start(2)

    g_wait(1)
    compute(197, 1)
    w_start(1)
    w_wait(3)
    i_wait(199, 3)
    decode(3)
    g_start(3)

    g_wait(2)
    compute(198, 2)
    w_start(2)

    g_wait(3)
    compute(199, 3)
    w_start(3)

    w_wait(0)
    w_wait(1)
    w_wait(2)
    w_wait(3)


@jax.jit
def _sc_call(enc_t, table, pe):
    mesh = plsc.VectorSubcoreMesh(core_axis_name="c", subcore_axis_name="s",
                                  num_cores=NC, num_subcores=NS)
    return pl.kernel(
        _sc_body,
        out_type=jax.ShapeDtypeStruct((FLAT, HIDDEN), jnp.float32),
        mesh=mesh,
        scratch_types=(
            [pltpu.VMEM((SEQ, HIDDEN), jnp.float32),
             pltpu.VMEM((NSEQ,), jnp.int32)]
            + [pltpu.VMEM((NSEQ, HIDDEN), jnp.float32)] * NB
            + [pltpu.VMEM((NSEQ,), jnp.int32)] * NB
            + [pltpu.VMEM((NSEQ,), jnp.float32)] * NB
            + [pltpu.VMEM((NSEQ,), jnp.int32)] * NB
            + [pltpu.SemaphoreType.DMA] * (3 * NB)
        ),
    )(enc_t, table, pe)


def kernel(inputs, mask, table):
    enc = inputs | (mask.astype(jnp.int32) << 31)
    enc_t = enc.reshape(NW, NSEQ, SEQ).transpose(0, 2, 1)
    pe = jnp.asarray(_PE)
    out = _sc_call(enc_t, table, pe)
    return out.reshape(BATCH, SEQ, HIDDEN)


# speedup vs baseline: 1.0094x; 1.0094x over previous
import jax
import jax.numpy as jnp
import numpy as np
from jax import lax
from jax.experimental import pallas as pl
from jax.experimental.pallas import tpu as pltpu
from jax.experimental.pallas import tpu_sc as plsc

HIDDEN = 128
N_SYMBOLS = 100000
BATCH = 4096
SEQ = 200

NC, NS, LANES = 2, 16, 16
NW = NC * NS
FLAT = BATCH * SEQ
PER_W = FLAT // NW
NSEQ = PER_W // SEQ
VREGS = HIDDEN // LANES


def _pe_table() -> np.ndarray:
    power = np.arange(0, HIDDEN, 2, dtype=np.float32) / np.float32(HIDDEN)
    divisor = np.float32(10000.0) ** power
    seq_pos = np.arange(SEQ, dtype=np.float32) + np.float32(1.0)
    arg = seq_pos[:, None] / divisor[None, :]
    pe = np.empty((SEQ, HIDDEN), dtype=np.float32)
    pe[:, 0::2] = np.sin(arg)
    pe[:, 1::2] = np.cos(arg)
    return pe


_PE = _pe_table()


def _sc_body(enc_hbm, table_hbm, pe_hbm, out_hbm,
             pe_v, ovec_v, r0, r1, r2, i0, i1, i2, m0, m1, m2, o0, o1, o2,
             gs0, gs1, gs2, ws0, ws1, ws2, qs0, qs1, qs2):
    rows = (r0, r1, r2)
    ibuf = (i0, i1, i2)
    mbuf = (m0, m1, m2)
    obuf = (o0, o1, o2)
    gsem = (gs0, gs1, gs2)
    wsem = (ws0, ws1, ws2)
    isem = (qs0, qs1, qs2)

    wid = lax.axis_index("s") * NC + lax.axis_index("c")
    wbase = wid * PER_W
    encw = enc_hbm.at[wid]

    pltpu.sync_copy(pe_hbm, pe_v)

    for u in range(VREGS):
        sl = pl.ds(u * LANES, LANES)
        ovec_v[sl] = (lax.iota(jnp.int32, LANES) + (u * LANES)) * SEQ + wbase

    def i_start(c, k):
        pltpu.async_copy(encw.at[c], ibuf[k], isem[k])

    def i_wait(c, k):
        pltpu.make_async_copy(encw.at[c], ibuf[k], isem[k]).wait()

    def decode(k):
        for u in range(VREGS):
            sl = pl.ds(u * LANES, LANES)
            e16 = ibuf[k][sl]
            mbuf[k][sl] = jnp.where(e16 < 0, jnp.float32(1), jnp.float32(0))
            ibuf[k][sl] = e16 & jnp.int32(0x7FFFFFFF)

    def g_start(k):
        pltpu.async_copy(table_hbm.at[ibuf[k]], rows[k], gsem[k])

    def g_wait(k):
        pltpu.make_async_copy(table_hbm.at[ibuf[k]], rows[k], gsem[k]).wait()

    def w_start(k):
        pltpu.async_copy(rows[k], out_hbm.at[obuf[k]], wsem[k])

    def w_wait(k):
        pltpu.make_async_copy(rows[k], out_hbm.at[obuf[k]], wsem[k]).wait()

    def compute(c, k):
        buf = rows[k]
        for u in range(VREGS):
            sl = pl.ds(u * LANES, LANES)
            obuf[k][sl] = ovec_v[sl] + c
        p = [pe_v[c, pl.ds(v * LANES, LANES)] for v in range(VREGS)]

        @pl.loop(0, NSEQ // LANES)
        def _grp(gr):
            m16 = mbuf[k][pl.ds(gr * LANES, LANES)]
            for j16 in range(LANES):
                j = gr * LANES + j16
                m = m16[j16]
                for v in range(VREGS):
                    sl = pl.ds(v * LANES, LANES)
                    buf[j, sl] = (buf[j, sl] + p[v]) * m

    for c0 in range(3):
        pltpu.sync_copy(encw.at[c0], ibuf[c0])
        decode(c0)
    g_start(0)
    g_start(1)

    g_wait(0)
    compute(0, 0)
    w_start(0)
    g_start(2)
    i_start(3, 0)

    g_wait(1)
    compute(1, 1)
    w_start(1)
    w_wait(0)
    i_wait(3, 0)
    decode(0)
    g_start(0)
    i_start(4, 1)

    @pl.loop(2, 197, step=3)
    def _main(go):
        for j in range(3):
            c = go + j
            k = (2 + j) % 3
            kn = (j + 1) % 3
            g_wait(k)
            compute(c, k)
            w_start(k)
            w_wait(kn)
            i_wait(c + 2, kn)
            decode(kn)
            g_start(kn)
            i_start(c + 3, k)

    g_wait(2)
    compute(197, 2)
    w_start(2)
    w_wait(1)
    i_wait(199, 1)
    decode(1)
    g_start(1)

    g_wait(0)
    compute(198, 0)
    w_start(0)

    g_wait(1)
    compute(199, 1)
    w_start(1)

    w_wait(2)
    w_wait(0)
    w_wait(1)


@jax.jit
def _sc_call(enc_t, table, pe):
    mesh = plsc.VectorSubcoreMesh(core_axis_name="c", subcore_axis_name="s",
                                  num_cores=NC, num_subcores=NS)
    return pl.kernel(
        _sc_body,
        out_type=jax.ShapeDtypeStruct((FLAT, HIDDEN), jnp.float32),
        mesh=mesh,
        scratch_types=[
            pltpu.VMEM((SEQ, HIDDEN), jnp.float32),
            pltpu.VMEM((NSEQ,), jnp.int32),
            pltpu.VMEM((NSEQ, HIDDEN), jnp.float32),
            pltpu.VMEM((NSEQ, HIDDEN), jnp.float32),
            pltpu.VMEM((NSEQ, HIDDEN), jnp.float32),
            pltpu.VMEM((NSEQ,), jnp.int32),
            pltpu.VMEM((NSEQ,), jnp.int32),
            pltpu.VMEM((NSEQ,), jnp.int32),
            pltpu.VMEM((NSEQ,), jnp.float32),
            pltpu.VMEM((NSEQ,), jnp.float32),
            pltpu.VMEM((NSEQ,), jnp.float32),
            pltpu.VMEM((NSEQ,), jnp.int32),
            pltpu.VMEM((NSEQ,), jnp.int32),
            pltpu.VMEM((NSEQ,), jnp.int32),
            pltpu.SemaphoreType.DMA,
            pltpu.SemaphoreType.DMA,
            pltpu.SemaphoreType.DMA,
            pltpu.SemaphoreType.DMA,
            pltpu.SemaphoreType.DMA,
            pltpu.SemaphoreType.DMA,
            pltpu.SemaphoreType.DMA,
            pltpu.SemaphoreType.DMA,
            pltpu.SemaphoreType.DMA,
        ],
    )(enc_t, table, pe)


def kernel(inputs, mask, table):
    enc = inputs | (mask.astype(jnp.int32) << 31)
    enc_t = enc.reshape(NW, NSEQ, SEQ).transpose(0, 2, 1)
    pe = jnp.asarray(_PE)
    out = _sc_call(enc_t, table, pe)
    return out.reshape(BATCH, SEQ, HIDDEN)
